# baseline (speedup 1.0000x reference)
import jax
from jax.experimental import pallas as pl
from jax.experimental.pallas import tpu as pltpu


_CHUNKS = 16


def _copy_kernel(x_ref, o_ref, sems):
    rows = x_ref.shape[0] // _CHUNKS
    copies = [
        pltpu.make_async_copy(
            x_ref.at[pl.ds(i * rows, rows)],
            o_ref.at[pl.ds(i * rows, rows)],
            sems.at[i],
        )
        for i in range(_CHUNKS)
    ]
    for c in copies:
        c.start()
    for c in copies:
        c.wait()


def kernel(x, s):
    total_rows = x.shape[0] * x.shape[1]
    x2 = x.reshape(total_rows, x.shape[2])
    out = pl.pallas_call(
        _copy_kernel,
        out_shape=jax.ShapeDtypeStruct(x2.shape, x.dtype),
        in_specs=[pl.BlockSpec(memory_space=pl.ANY)],
        out_specs=pl.BlockSpec(memory_space=pl.ANY),
        scratch_shapes=[pltpu.SemaphoreType.DMA((_CHUNKS,))],
    )(x2)
    return out.reshape(x.shape)


# speedup vs baseline: 48.1203x; 48.1203x over previous
import jax
from jax.experimental import pallas as pl


_ROWS = 512


def _copy_kernel(x_ref, o_ref):
    o_ref[...] = x_ref[...]


def kernel(x, s):
    total_rows = x.shape[0] * x.shape[1]
    x2 = x.reshape(total_rows, x.shape[2])
    out = pl.pallas_call(
        _copy_kernel,
        grid=(total_rows // _ROWS,),
        in_specs=[pl.BlockSpec((_ROWS, x.shape[2]), lambda i: (i, 0))],
        out_specs=pl.BlockSpec((_ROWS, x.shape[2]), lambda i: (i, 0)),
        out_shape=jax.ShapeDtypeStruct(x2.shape, x.dtype),
    )(x2)
    return out.reshape(x.shape)


# speedup vs baseline: 49.2736x; 1.0240x over previous
import jax
from jax.experimental import pallas as pl
from jax.experimental.pallas import tpu as pltpu


_ROWS = 2040


def _copy_kernel(x_ref, o_ref):
    o_ref[...] = x_ref[...]


def kernel(x, s):
    total_rows = x.shape[0] * x.shape[1]
    x2 = x.reshape(total_rows, x.shape[2])
    out = pl.pallas_call(
        _copy_kernel,
        grid=((total_rows + _ROWS - 1) // _ROWS,),
        in_specs=[pl.BlockSpec((_ROWS, x.shape[2]), lambda i: (i, 0))],
        out_specs=pl.BlockSpec((_ROWS, x.shape[2]), lambda i: (i, 0)),
        out_shape=jax.ShapeDtypeStruct(x2.shape, x.dtype),
        compiler_params=pltpu.CompilerParams(
            vmem_limit_bytes=100 * 1024 * 1024,
        ),
    )(x2)
    return out.reshape(x.shape)
